# baseline (speedup 1.0000x reference)
import functools

import jax
import jax.numpy as jnp
from jax import lax
from jax.experimental import pallas as pl
from jax.experimental.pallas import tpu as pltpu
from jax.experimental.pallas import tpu_sc as plsc

_ROWS = 16384


_DN_T = (((0,), (1,)), ((), ()))


def _proj_body(ut, it, ct, at_, gt, wu, wi, wc, wa, wg, pu, pi_, pc, pa, pg):
    f32, bf16 = jnp.float32, jnp.bfloat16
    pu[...] = lax.dot_general(wu[...].astype(bf16), ut[...].astype(bf16),
                              _DN_T, preferred_element_type=f32)[0]
    pi_[...] = lax.dot_general(wi[...].astype(bf16), it[...].astype(bf16),
                               _DN_T, preferred_element_type=f32)[0]
    pc[...] = lax.dot_general(wc[...].astype(bf16), ct[...].astype(bf16),
                              _DN_T, preferred_element_type=f32)[0]

    @pl.when(pl.program_id(0) == 0)
    def _():
        pa[...] = lax.dot_general(wa[...], at_[...], _DN_T, preferred_element_type=f32)
        pg[...] = lax.dot_general(wg[...], gt[...], _DN_T, preferred_element_type=f32)


def _project(user_table, item_table, context_table, age_table, gender_table,
             wu, wi, wc, wa, wg):
    v = user_table.shape[0]
    grid = (v + _ROWS - 1) // _ROWS
    emb = user_table.shape[1]
    side = age_table.shape[1]
    na, ng = age_table.shape[0], gender_table.shape[0]
    big = pl.BlockSpec((_ROWS, emb), lambda i: (i, 0))
    whole = lambda s: pl.BlockSpec(s, lambda i: (0, 0))
    return pl.pallas_call(
        _proj_body,
        grid=(grid,),
        in_specs=[
            big, big, big,
            whole((na, side)), whole((ng, side)),
            whole((emb, 1)), whole((emb, 1)), whole((emb, 1)),
            whole((side, 1)), whole((side, 1)),
        ],
        out_specs=[
            pl.BlockSpec((_ROWS,), lambda i: (i,)),
            pl.BlockSpec((_ROWS,), lambda i: (i,)),
            pl.BlockSpec((_ROWS,), lambda i: (i,)),
            whole((1, na)), whole((1, ng)),
        ],
        out_shape=[
            jax.ShapeDtypeStruct((grid * _ROWS,), jnp.float32),
            jax.ShapeDtypeStruct((grid * _ROWS,), jnp.float32),
            jax.ShapeDtypeStruct((grid * _ROWS,), jnp.float32),
            jax.ShapeDtypeStruct((1, na), jnp.float32),
            jax.ShapeDtypeStruct((1, ng), jnp.float32),
        ],
    )(user_table, item_table, context_table, age_table, gender_table,
      wu, wi, wc, wa, wg)


_NW = 32
_L = 16
_CTX = 20


def _sc_body(vocab, b_per_w,
             pu_h, pi_h, pc_h, pa_h, pg_h, uidx_h, iidx_h, aidx_h, gidx_h,
             cidx_h, upc_h, wb_h, out_h,
             uidx_v, iidx_v, aidx_v, gidx_v, upc_v, cidx_v, cvals,
             puv, piv, pa_t, pg_t, wb_v, out_v, pc_sh, sem, csem):
    wid = lax.axis_index("s") * 2 + lax.axis_index("c")
    base = wid * b_per_w
    pltpu.sync_copy(uidx_h.at[pl.ds(base, b_per_w)], uidx_v)
    pltpu.sync_copy(iidx_h.at[pl.ds(base, b_per_w)], iidx_v)
    d1 = pltpu.async_copy(pu_h.at[uidx_v], puv, sem)
    d2 = pltpu.async_copy(pi_h.at[iidx_v], piv, sem)
    stg = [
        pltpu.async_copy(cidx_h.at[pl.ds(base * _CTX, b_per_w * _CTX)], cidx_v, csem),
        pltpu.async_copy(aidx_h.at[pl.ds(base, b_per_w)], aidx_v, csem),
        pltpu.async_copy(gidx_h.at[pl.ds(base, b_per_w)], gidx_v, csem),
        pltpu.async_copy(upc_h.at[pl.ds(base, b_per_w)], upc_v, csem),
        pltpu.async_copy(pa_h, pa_t, csem),
        pltpu.async_copy(pg_h, pg_t, csem),
        pltpu.async_copy(wb_h, wb_v, csem),
    ]
    @pl.when(lax.axis_index("s") == 0)
    def _():
        pltpu.sync_copy(pc_h, pc_sh)
    for d in stg:
        d.wait()
    plsc.subcore_barrier()
    dc = pltpu.async_copy(pc_sh.at[cidx_v], cvals, csem)
    dc.wait()
    d1.wait()
    d2.wait()

    wt = wb_v[pl.ds(0, _L)]
    bv = wb_v[pl.ds(_L, _L)]
    lane = lax.iota(jnp.int32, _L)
    nchunk = b_per_w // _L
    for c in range(nchunk):
        s = c * _L
        acc = upc_v[pl.ds(s, _L)] * wt + bv
        acc = acc + puv[pl.ds(s, _L)]
        acc = acc + piv[pl.ds(s, _L)]
        acc = acc + plsc.load_gather(pa_t, [aidx_v[pl.ds(s, _L)]])
        acc = acc + plsc.load_gather(pg_t, [gidx_v[pl.ds(s, _L)]])
        cbase = lane * _CTX + s * _CTX
        for j in range(_CTX):
            acc = acc + plsc.load_gather(cvals, [cbase + j])
        out_v[pl.ds(s, _L)] = acc
    pltpu.sync_copy(out_v, out_h.at[pl.ds(base, b_per_w)])


def _sc_lookup(pu, pi, pc, pa, pg, uidx, iidx, aidx, gidx, cidx, upc, wb):
    b = uidx.shape[0]
    b_per_w = b // _NW
    vocab = pc.shape[0]
    mesh = plsc.VectorSubcoreMesh(core_axis_name="c", subcore_axis_name="s")
    f32, i32 = jnp.float32, jnp.int32
    kern = functools.partial(
        pl.kernel,
        mesh=mesh,
        compiler_params=pltpu.CompilerParams(needs_layout_passes=False),
        out_type=jax.ShapeDtypeStruct((b,), f32),
        scratch_types=[
            pltpu.VMEM((b_per_w,), i32),
            pltpu.VMEM((b_per_w,), i32),
            pltpu.VMEM((b_per_w,), i32),
            pltpu.VMEM((b_per_w,), i32),
            pltpu.VMEM((b_per_w,), f32),
            pltpu.VMEM((b_per_w * _CTX,), i32),
            pltpu.VMEM((b_per_w * _CTX,), f32),
            pltpu.VMEM((b_per_w,), f32),
            pltpu.VMEM((b_per_w,), f32),
            pltpu.VMEM((pa.shape[0],), f32),
            pltpu.VMEM((pg.shape[0],), f32),
            pltpu.VMEM((wb.shape[0],), f32),
            pltpu.VMEM((b_per_w,), f32),
            pltpu.VMEM_SHARED((vocab,), f32),
            pltpu.SemaphoreType.DMA,
            pltpu.SemaphoreType.DMA,
        ],
    )(functools.partial(_sc_body, vocab, b_per_w))
    return kern(pu, pi, pc, pa, pg, uidx, iidx, aidx, gidx, cidx, upc, wb)


def kernel(user_idx, gender, age, context_idx, item_idx, user_product_count,
           user_table, gender_table, age_table, context_table, item_table, W, b):
    emb = user_table.shape[1]
    side = gender_table.shape[1]
    bsz = user_idx.shape[0]
    o0, o1, o2, o3, o4 = emb, emb + side, emb + 2 * side, 2 * emb + 2 * side, 3 * emb + 2 * side
    wu = W[:o0]
    wg = W[o0:o1]
    wa = W[o1:o2]
    wc = W[o2:o3] * (5.0 / context_idx.shape[1])
    wi = W[o3:o4]
    pu2, pi2, pc2, pa2, pg2 = _project(
        user_table, item_table, context_table, age_table, gender_table,
        wu, wi, wc, wa, wg)
    pa = jnp.pad(pa2[0], (0, 128 - pa2.shape[1]))
    pg = jnp.pad(pg2[0], (0, 16 - pg2.shape[1]))
    wb = jnp.concatenate([
        jnp.broadcast_to(W[o4, 0], (16,)),
        jnp.broadcast_to(b[0], (16,)),
    ]).astype(jnp.float32)
    out1 = _sc_lookup(
        pu2, pi2, pc2, pa, pg,
        user_idx, item_idx, age, gender,
        context_idx.reshape(-1), user_product_count, wb)
    return out1.reshape(bsz, 1)


# speedup vs baseline: 1.0133x; 1.0133x over previous
import functools

import jax
import jax.numpy as jnp
from jax import lax
from jax.experimental import pallas as pl
from jax.experimental.pallas import tpu as pltpu
from jax.experimental.pallas import tpu_sc as plsc

_ROWS = 8192


_DN_T = (((0,), (1,)), ((), ()))


def _proj_body(ut, it, ct, at_, gt, wu, wi, wc, wa, wg, pu, pi_, pc, pa, pg):
    f32, bf16 = jnp.float32, jnp.bfloat16
    pu[...] = lax.dot_general(wu[...].astype(bf16), ut[...].astype(bf16),
                              _DN_T, preferred_element_type=f32)[0]
    pi_[...] = lax.dot_general(wi[...].astype(bf16), it[...].astype(bf16),
                               _DN_T, preferred_element_type=f32)[0]
    pc[...] = lax.dot_general(wc[...].astype(bf16), ct[...].astype(bf16),
                              _DN_T, preferred_element_type=f32)[0]

    @pl.when(pl.program_id(0) == 0)
    def _():
        pa[...] = lax.dot_general(wa[...], at_[...], _DN_T, preferred_element_type=f32)
        pg[...] = lax.dot_general(wg[...], gt[...], _DN_T, preferred_element_type=f32)


def _project(user_table, item_table, context_table, age_table, gender_table,
             wu, wi, wc, wa, wg):
    v = user_table.shape[0]
    grid = (v + _ROWS - 1) // _ROWS
    emb = user_table.shape[1]
    side = age_table.shape[1]
    na, ng = age_table.shape[0], gender_table.shape[0]
    big = pl.BlockSpec((_ROWS, emb), lambda i: (i, 0))
    whole = lambda s: pl.BlockSpec(s, lambda i: (0, 0))
    return pl.pallas_call(
        _proj_body,
        grid=(grid,),
        in_specs=[
            big, big, big,
            whole((na, side)), whole((ng, side)),
            whole((emb, 1)), whole((emb, 1)), whole((emb, 1)),
            whole((side, 1)), whole((side, 1)),
        ],
        out_specs=[
            pl.BlockSpec((_ROWS,), lambda i: (i,)),
            pl.BlockSpec((_ROWS,), lambda i: (i,)),
            pl.BlockSpec((_ROWS,), lambda i: (i,)),
            whole((1, na)), whole((1, ng)),
        ],
        out_shape=[
            jax.ShapeDtypeStruct((grid * _ROWS,), jnp.float32),
            jax.ShapeDtypeStruct((grid * _ROWS,), jnp.float32),
            jax.ShapeDtypeStruct((grid * _ROWS,), jnp.float32),
            jax.ShapeDtypeStruct((1, na), jnp.float32),
            jax.ShapeDtypeStruct((1, ng), jnp.float32),
        ],
    )(user_table, item_table, context_table, age_table, gender_table,
      wu, wi, wc, wa, wg)


_NW = 32
_L = 16
_CTX = 20


def _sc_body(vocab, b_per_w,
             pu_h, pi_h, pc_h, pa_h, pg_h, uidx_h, iidx_h, aidx_h, gidx_h,
             cidx_h, upc_h, wb_h, out_h,
             uidx_v, iidx_v, aidx_v, gidx_v, upc_v, cidx_v, cvals,
             puv, piv, pa_t, pg_t, wb_v, out_v, pc_sh, sem, csem):
    wid = lax.axis_index("s") * 2 + lax.axis_index("c")
    base = wid * b_per_w
    pltpu.sync_copy(uidx_h.at[pl.ds(base, b_per_w)], uidx_v)
    pltpu.sync_copy(iidx_h.at[pl.ds(base, b_per_w)], iidx_v)
    d1 = pltpu.async_copy(pu_h.at[uidx_v], puv, sem)
    d2 = pltpu.async_copy(pi_h.at[iidx_v], piv, sem)
    stg = [
        pltpu.async_copy(cidx_h.at[pl.ds(base * _CTX, b_per_w * _CTX)], cidx_v, csem),
        pltpu.async_copy(aidx_h.at[pl.ds(base, b_per_w)], aidx_v, csem),
        pltpu.async_copy(gidx_h.at[pl.ds(base, b_per_w)], gidx_v, csem),
        pltpu.async_copy(upc_h.at[pl.ds(base, b_per_w)], upc_v, csem),
        pltpu.async_copy(pa_h, pa_t, csem),
        pltpu.async_copy(pg_h, pg_t, csem),
        pltpu.async_copy(wb_h, wb_v, csem),
    ]
    @pl.when(lax.axis_index("s") == 0)
    def _():
        pltpu.sync_copy(pc_h, pc_sh)
    for d in stg:
        d.wait()
    plsc.subcore_barrier()
    dc = pltpu.async_copy(pc_sh.at[cidx_v], cvals, csem)
    dc.wait()
    d1.wait()
    d2.wait()

    wt = wb_v[pl.ds(0, _L)]
    bv = wb_v[pl.ds(_L, _L)]
    lane = lax.iota(jnp.int32, _L)
    nchunk = b_per_w // _L
    for c in range(nchunk):
        s = c * _L
        acc = upc_v[pl.ds(s, _L)] * wt + bv
        acc = acc + puv[pl.ds(s, _L)]
        acc = acc + piv[pl.ds(s, _L)]
        acc = acc + plsc.load_gather(pa_t, [aidx_v[pl.ds(s, _L)]])
        acc = acc + plsc.load_gather(pg_t, [gidx_v[pl.ds(s, _L)]])
        cbase = lane * _CTX + s * _CTX
        for j in range(_CTX):
            acc = acc + plsc.load_gather(cvals, [cbase + j])
        out_v[pl.ds(s, _L)] = acc
    pltpu.sync_copy(out_v, out_h.at[pl.ds(base, b_per_w)])


def _sc_lookup(pu, pi, pc, pa, pg, uidx, iidx, aidx, gidx, cidx, upc, wb):
    b = uidx.shape[0]
    b_per_w = b // _NW
    vocab = pc.shape[0]
    mesh = plsc.VectorSubcoreMesh(core_axis_name="c", subcore_axis_name="s")
    f32, i32 = jnp.float32, jnp.int32
    kern = functools.partial(
        pl.kernel,
        mesh=mesh,
        compiler_params=pltpu.CompilerParams(needs_layout_passes=False),
        out_type=jax.ShapeDtypeStruct((b,), f32),
        scratch_types=[
            pltpu.VMEM((b_per_w,), i32),
            pltpu.VMEM((b_per_w,), i32),
            pltpu.VMEM((b_per_w,), i32),
            pltpu.VMEM((b_per_w,), i32),
            pltpu.VMEM((b_per_w,), f32),
            pltpu.VMEM((b_per_w * _CTX,), i32),
            pltpu.VMEM((b_per_w * _CTX,), f32),
            pltpu.VMEM((b_per_w,), f32),
            pltpu.VMEM((b_per_w,), f32),
            pltpu.VMEM((pa.shape[0],), f32),
            pltpu.VMEM((pg.shape[0],), f32),
            pltpu.VMEM((wb.shape[0],), f32),
            pltpu.VMEM((b_per_w,), f32),
            pltpu.VMEM_SHARED((vocab,), f32),
            pltpu.SemaphoreType.DMA,
            pltpu.SemaphoreType.DMA,
        ],
    )(functools.partial(_sc_body, vocab, b_per_w))
    return kern(pu, pi, pc, pa, pg, uidx, iidx, aidx, gidx, cidx, upc, wb)


def kernel(user_idx, gender, age, context_idx, item_idx, user_product_count,
           user_table, gender_table, age_table, context_table, item_table, W, b):
    emb = user_table.shape[1]
    side = gender_table.shape[1]
    bsz = user_idx.shape[0]
    o0, o1, o2, o3, o4 = emb, emb + side, emb + 2 * side, 2 * emb + 2 * side, 3 * emb + 2 * side
    wu = W[:o0]
    wg = W[o0:o1]
    wa = W[o1:o2]
    wc = W[o2:o3] * (5.0 / context_idx.shape[1])
    wi = W[o3:o4]
    pu2, pi2, pc2, pa2, pg2 = _project(
        user_table, item_table, context_table, age_table, gender_table,
        wu, wi, wc, wa, wg)
    pa = jnp.pad(pa2[0], (0, 128 - pa2.shape[1]))
    pg = jnp.pad(pg2[0], (0, 16 - pg2.shape[1]))
    wb = jnp.concatenate([
        jnp.broadcast_to(W[o4, 0], (16,)),
        jnp.broadcast_to(b[0], (16,)),
    ]).astype(jnp.float32)
    out1 = _sc_lookup(
        pu2, pi2, pc2, pa, pg,
        user_idx, item_idx, age, gender,
        context_idx.reshape(-1), user_product_count, wb)
    return out1.reshape(bsz, 1)
